# baseline (speedup 1.0000x reference)
import functools

import jax
import jax.numpy as jnp
from jax import lax
from jax.experimental import pallas as pl
from jax.experimental.pallas import tpu as pltpu
from jax.experimental.pallas import tpu_sc as plsc

_NC = 2
_NS = 16


@functools.lru_cache(maxsize=None)
def _make_gather(B, D):
    NW = _NC * _NS
    RPW = B // NW
    CH = RPW // 2
    mesh = plsc.VectorSubcoreMesh(core_axis_name="c", subcore_axis_name="s")

    @functools.partial(
        pl.kernel,
        out_type=(jax.ShapeDtypeStruct((B, D), jnp.float32),
                  jax.ShapeDtypeStruct((B, D), jnp.float32)),
        mesh=mesh,
        scratch_types=[
            pltpu.VMEM((RPW,), jnp.int32),
            pltpu.VMEM((RPW,), jnp.int32),
            pltpu.VMEM((CH, D), jnp.float32),
            pltpu.VMEM((CH, D), jnp.float32),
            pltpu.SemaphoreType.DMA,
            pltpu.SemaphoreType.DMA,
            pltpu.SemaphoreType.DMA,
            pltpu.SemaphoreType.DMA,
        ],
    )
    def gather_k(name_tab, dom_tab, name_idx, dom_idx, ne_out, de_out,
                 idx_n, idx_d, buf0, buf1, sg0, sg1, sw0, sw1):
        wid = lax.axis_index("s") * _NC + lax.axis_index("c")
        base = wid * RPW
        pltpu.sync_copy(name_idx.at[pl.ds(base, RPW)], idx_n)
        pltpu.sync_copy(dom_idx.at[pl.ds(base, RPW)], idx_d)

        def gather(tab, idx_v, c, buf, sem):
            return pltpu.async_copy(
                tab.at[idx_v.at[pl.ds(c * CH, CH)]], buf, sem)

        def write(out, c, buf, sem):
            return pltpu.async_copy(
                buf, out.at[pl.ds(base + c * CH, CH)], sem)

        g0 = gather(name_tab, idx_n, 0, buf0, sg0)
        g1 = gather(name_tab, idx_n, 1, buf1, sg1)
        g0.wait()
        w0 = write(ne_out, 0, buf0, sw0)
        g1.wait()
        w1 = write(ne_out, 1, buf1, sw1)
        w0.wait()
        g2 = gather(dom_tab, idx_d, 0, buf0, sg0)
        w1.wait()
        g3 = gather(dom_tab, idx_d, 1, buf1, sg1)
        g2.wait()
        w2 = write(de_out, 0, buf0, sw0)
        g3.wait()
        w3 = write(de_out, 1, buf1, sw1)
        w2.wait()
        w3.wait()

    return gather_k


def _mlp_body(ne_ref, de_ref, w1a_ref, w1b_ref, b1_ref, w2_ref, b2_ref,
              out_ref):
    h = jnp.dot(ne_ref[...], w1a_ref[...], preferred_element_type=jnp.float32)
    h = h + jnp.dot(de_ref[...], w1b_ref[...],
                    preferred_element_type=jnp.float32)
    h = jnp.maximum(h + b1_ref[...], 0.0)
    v = jnp.sum(h * w2_ref[...], axis=1) + b2_ref[0]
    out_ref[...] = 5.0 * jax.nn.sigmoid(v)


@functools.lru_cache(maxsize=None)
def _make_mlp(B, D, H, BLK):
    grid = (B // BLK,)
    return pl.pallas_call(
        _mlp_body,
        grid=grid,
        in_specs=[
            pl.BlockSpec((BLK, D), lambda i: (i, 0)),
            pl.BlockSpec((BLK, D), lambda i: (i, 0)),
            pl.BlockSpec((D, H), lambda i: (0, 0)),
            pl.BlockSpec((D, H), lambda i: (0, 0)),
            pl.BlockSpec((1, H), lambda i: (0, 0)),
            pl.BlockSpec((1, H), lambda i: (0, 0)),
            pl.BlockSpec(memory_space=pltpu.SMEM),
        ],
        out_specs=pl.BlockSpec((BLK,), lambda i: (i,)),
        out_shape=jax.ShapeDtypeStruct((B,), jnp.float32),
    )


def kernel(name_indices, domain_indices, name_table, domain_table,
           W1, b1, W2, b2):
    B = name_indices.shape[0]
    D = name_table.shape[1]
    H = W1.shape[1]
    ne, de = _make_gather(B, D)(
        name_table, domain_table,
        name_indices.astype(jnp.int32), domain_indices.astype(jnp.int32))
    out = _make_mlp(B, D, H, 1024)(
        ne, de, W1[:D], W1[D:], b1.reshape(1, H), W2.reshape(1, H), b2)
    return out
